# Initial kernel scaffold; baseline (speedup 1.0000x reference)
import jax
import jax.numpy as jnp
from jax.experimental import pallas as pl


def kernel(x, edge_index, edge_attr, batch, classification_model, colors, train, We1, W1a, b1a, W1b, b1b, We2, W2a, b2a, W2b, b2b, We3, W3a, b3a, W3b, b3b):
    raise NotImplementedError("write your pallas kernel here")


# speedup vs baseline: 1.0000x; 1.0000x over previous
import functools

import jax
import jax.numpy as jnp
from jax.experimental import pallas as pl

_N = 10000
_NG = 128
_NCOLORS = 4
_NITER = 5


def _mask_body(s_ref, o_ref):
    o_ref[...] = s_ref[...] > 0.0


def _gine(x, src, dst, e, We, Wa, ba, Wb, bb):
    msg = jax.nn.relu(x[src] + e @ We)
    agg = jax.ops.segment_sum(msg, dst, num_segments=x.shape[0])
    h = x + agg
    return jax.nn.relu(h @ Wa + ba) @ Wb + bb


def kernel(x, edge_index, edge_attr, batch, classification_model, colors, train,
           We1, W1a, b1a, W1b, b1b,
           We2, W2a, b2a, W2b, b2b,
           We3, W3a, b3a, W3b, b3b):
    src = edge_index[0]
    dst = edge_index[1]
    h = _gine(x, src, dst, edge_attr, We1, W1a, b1a, W1b, b1b)
    h = _gine(h, src, dst, edge_attr, We2, W2a, b2a, W2b, b2b)
    h = _gine(h, src, dst, edge_attr, We3, W3a, b3a, W3b, b3b)
    sums = jax.ops.segment_sum(h, batch, num_segments=_NG)
    cnts = jax.ops.segment_sum(jnp.ones((h.shape[0], 1), dtype=h.dtype), batch,
                               num_segments=_NG)
    mean = sums / jnp.clip(cnts, 1.0)
    h = h - mean[batch]
    hf = h.flatten()
    T = 0.1
    key = jax.random.key(7)
    spins = jnp.where(jax.random.uniform(key, (hf.shape[0],)) < 0.5, 1.0, -1.0).astype(jnp.float32)
    for it in range(_NITER):
        for c in range(_NCOLORS):
            k = jax.random.fold_in(key, it * _NCOLORS + c + 1)
            nb = jax.ops.segment_sum(spins[src], dst, num_segments=hf.shape[0])
            field = nb + hf
            p = jax.nn.sigmoid(2.0 * field / T)
            new = jnp.where(jax.random.uniform(k, (hf.shape[0],)) < p, 1.0, -1.0).astype(jnp.float32)
            spins = jnp.where(colors == c, new, spins)
    mask2d = pl.pallas_call(
        _mask_body,
        out_shape=jax.ShapeDtypeStruct((100, 100), jnp.bool_),
    )(spins.reshape(100, 100))
    return (mask2d.reshape(-1), hf)


# speedup vs baseline: 13.6166x; 13.6165x over previous
import jax
import jax.numpy as jnp
from jax import lax
from jax.experimental import pallas as pl
from jax.experimental.pallas import tpu as pltpu
from jax.experimental.pallas import tpu_sc as plsc

_N = 10000
_E = 640000
_NG = 128
_NCOLORS = 4
_NITER = 5
_NSTEPS = _NITER * _NCOLORS

_NP = 10240
_TILES = 16
_EPT = _E // _TILES
_NPT = _NP // _TILES
_NROWS = _NP // 16
_RPT = _NROWS // _TILES
_BROWS = 32
_BPAD = 256


def _ising_body(src_hbm, dst_hbm, h_hbm, batch_hbm, colors_hbm, spins0_hbm,
                u_hbm, rowids_hbm, binids_hbm, zrows_hbm, zbins_hbm, znb_hbm,
                hf_out, spins_out,
                src_v, dst_v, spins_v, nb_v, h_v, batch_v, colors_v, hf_v,
                u_v, bins_v, binsred_v, mean_v, nbsl_v, rowids_v, binids_v,
                zrows_v, shared_nb, shared_sp, shared_bins):
    cid = lax.axis_index("c")
    sid = lax.axis_index("s")

    @pl.when(cid == 0)
    def _work():
        ebase = sid * _EPT
        nbase = sid * _NPT
        rbase = sid * _RPT
        lanes = lax.iota(jnp.int32, 16)

        pltpu.sync_copy(src_hbm.at[pl.ds(ebase, _EPT)], src_v)
        pltpu.sync_copy(dst_hbm.at[pl.ds(ebase, _EPT)], dst_v)
        pltpu.sync_copy(h_hbm.at[pl.ds(nbase, _NPT)], h_v)
        pltpu.sync_copy(batch_hbm.at[pl.ds(nbase, _NPT)], batch_v)
        pltpu.sync_copy(colors_hbm.at[pl.ds(nbase, _NPT)], colors_v)
        pltpu.sync_copy(spins0_hbm, spins_v)
        pltpu.sync_copy(rowids_hbm, rowids_v)
        pltpu.sync_copy(binids_hbm, binids_v)
        pltpu.sync_copy(zrows_hbm, zrows_v)
        pltpu.sync_copy(zbins_hbm, bins_v)
        for s in range(_NSTEPS):
            pltpu.sync_copy(u_hbm.at[pl.ds(s * _NP + nbase, _NPT)],
                            u_v.at[pl.ds(s * _NPT, _NPT)])

        ones = jnp.ones((16,), jnp.float32)

        def _rb(i, carry):
            b = batch_v[pl.ds(i * 16, 16)]
            hv = h_v[pl.ds(i * 16, 16)]
            hi = jnp.right_shift(b, 4)
            lo = jnp.bitwise_and(b, 15)
            plsc.addupdate_scatter(bins_v, [hi, lo], hv)
            plsc.addupdate_scatter(bins_v, [hi + 8, lo], ones)
            return carry
        lax.fori_loop(0, _NPT // 16, _rb, 0)

        @pl.when(sid == 0)
        def _zb():
            pltpu.sync_copy(zbins_hbm, shared_bins)
        plsc.subcore_barrier()
        pltpu.sync_copy(bins_v, shared_bins.at[binids_v.at[0]], add=True)
        plsc.subcore_barrier()
        pltpu.sync_copy(shared_bins, binsred_v)

        for i in range(_NG // 16):
            ri = jnp.full((16,), i, jnp.int32)
            sb = plsc.load_gather(binsred_v, [ri, lanes])
            cb = plsc.load_gather(binsred_v, [ri + 8, lanes])
            m = sb / jnp.maximum(cb, 1.0)
            plsc.store_scatter(mean_v, [ri, lanes], m)

        def _hb(i, carry):
            b = jnp.minimum(batch_v[pl.ds(i * 16, 16)], 127)
            m = plsc.load_gather(mean_v,
                                 [jnp.right_shift(b, 4),
                                  jnp.bitwise_and(b, 15)])
            hf_v[pl.ds(i * 16, 16)] = h_v[pl.ds(i * 16, 16)] - m
            return carry
        lax.fori_loop(0, _NPT // 16, _hb, 0)
        pltpu.sync_copy(hf_v, hf_out.at[pl.ds(nbase, _NPT)])

        for s in range(_NSTEPS):
            c = s % _NCOLORS
            pltpu.sync_copy(znb_hbm, nb_v)
            pltpu.sync_copy(zrows_v, shared_nb.at[pl.ds(rbase, _RPT)])
            plsc.subcore_barrier()

            def _eb(i, carry):
                sv = src_v[pl.ds(i * 16, 16)]
                dv = dst_v[pl.ds(i * 16, 16)]
                sp = plsc.load_gather(spins_v,
                                      [jnp.right_shift(sv, 4),
                                       jnp.bitwise_and(sv, 15)])
                plsc.addupdate_scatter(
                    nb_v, [jnp.right_shift(dv, 4), jnp.bitwise_and(dv, 15)],
                    sp)
                return carry
            lax.fori_loop(0, _EPT // 16, _eb, 0)

            for j in range(_NROWS // 128):
                pltpu.sync_copy(nb_v.at[pl.ds(j * 128, 128)],
                                shared_nb.at[rowids_v.at[j]], add=True)
            plsc.subcore_barrier()

            pltpu.sync_copy(shared_nb.at[pl.ds(rbase, _RPT)], nbsl_v)

            def _ub(i, carry):
                ri = jnp.full((16,), i, jnp.int32)
                nb16 = plsc.load_gather(nbsl_v, [ri, lanes])
                f = nb16 + hf_v[pl.ds(i * 16, 16)]
                p = 1.0 / (1.0 + jnp.exp(f * -20.0))
                u = u_v[pl.ds(s * _NPT + i * 16, 16)]
                newsp = jnp.where(u < p, 1.0, -1.0)
                col = colors_v[pl.ds(i * 16, 16)]
                grow = jnp.full((16,), rbase + i, jnp.int32)
                old = plsc.load_gather(spins_v, [grow, lanes])
                plsc.store_scatter(spins_v, [grow, lanes],
                                   jnp.where(col == c, newsp, old))
                return carry
            lax.fori_loop(0, _NPT // 16, _ub, 0)

            pltpu.sync_copy(spins_v.at[pl.ds(rbase, _RPT)],
                            shared_sp.at[pl.ds(rbase, _RPT)])
            plsc.subcore_barrier()
            pltpu.sync_copy(shared_sp, spins_v)

        pltpu.sync_copy(spins_v.at[pl.ds(rbase, _RPT)],
                        spins_out.at[pl.ds(rbase, _RPT)])


def _make_ising_call(interpret=False):
  return pl.kernel(
    _ising_body,
    out_type=(jax.ShapeDtypeStruct((_NP,), jnp.float32),
              jax.ShapeDtypeStruct((_NROWS, 16), jnp.float32)),
    mesh=plsc.VectorSubcoreMesh(core_axis_name="c", subcore_axis_name="s",
                                num_cores=2, num_subcores=16),
    interpret=interpret,
    compiler_params=pltpu.CompilerParams(needs_layout_passes=False,
                                         use_tc_tiling_on_sc=False),
    scratch_types=[
        pltpu.VMEM((_EPT,), jnp.int32),
        pltpu.VMEM((_EPT,), jnp.int32),
        pltpu.VMEM((_NROWS, 16), jnp.float32),
        pltpu.VMEM((_NROWS, 16), jnp.float32),
        pltpu.VMEM((_NPT,), jnp.float32),
        pltpu.VMEM((_NPT,), jnp.int32),
        pltpu.VMEM((_NPT,), jnp.int32),
        pltpu.VMEM((_NPT,), jnp.float32),
        pltpu.VMEM((_NSTEPS * _NPT,), jnp.float32),
        pltpu.VMEM((_BROWS, 16), jnp.float32),
        pltpu.VMEM((_BROWS, 16), jnp.float32),
        pltpu.VMEM((_NG // 16, 16), jnp.float32),
        pltpu.VMEM((_RPT, 16), jnp.float32),
        pltpu.VMEM((_NROWS // 128, 128), jnp.int32),
        pltpu.VMEM((1, _BROWS), jnp.int32),
        pltpu.VMEM((_RPT, 16), jnp.float32),
        pltpu.VMEM_SHARED((_NROWS, 16), jnp.float32),
        pltpu.VMEM_SHARED((_NROWS, 16), jnp.float32),
        pltpu.VMEM_SHARED((_BROWS, 16), jnp.float32),
    ],
  )


_ising_call = _make_ising_call()


def _gine(x, src, dst, e, We, Wa, ba, Wb, bb):
    msg = jax.nn.relu(x[src] + e @ We)
    agg = jax.ops.segment_sum(msg, dst, num_segments=x.shape[0])
    h = x + agg
    return jax.nn.relu(h @ Wa + ba) @ Wb + bb


def kernel(x, edge_index, edge_attr, batch, classification_model, colors, train,
           We1, W1a, b1a, W1b, b1b,
           We2, W2a, b2a, W2b, b2b,
           We3, W3a, b3a, W3b, b3b):
    src = edge_index[0]
    dst = edge_index[1]
    h = _gine(x, src, dst, edge_attr, We1, W1a, b1a, W1b, b1b)
    h = _gine(h, src, dst, edge_attr, We2, W2a, b2a, W2b, b2b)
    h = _gine(h, src, dst, edge_attr, We3, W3a, b3a, W3b, b3b)
    hflat = h.reshape(-1)

    key = jax.random.key(7)
    u0 = jax.random.uniform(key, (_N,))
    spins0 = jnp.where(u0 < 0.5, 1.0, -1.0).astype(jnp.float32)
    urows = [jax.random.uniform(jax.random.fold_in(key, s + 1), (_N,))
             for s in range(_NSTEPS)]
    pad = _NP - _N
    u_flat = jnp.concatenate(
        [jnp.pad(r, (0, pad)) for r in urows]).astype(jnp.float32)

    h_pad = jnp.pad(hflat, (0, pad))
    batch_pad = jnp.pad(batch, (0, pad), constant_values=_BPAD)
    colors_pad = jnp.pad(colors, (0, pad), constant_values=-1)
    spins0_pad = jnp.pad(spins0, (0, pad)).reshape(_NROWS, 16)
    rowids = jnp.arange(_NROWS, dtype=jnp.int32).reshape(_NROWS // 128, 128)
    binids = jnp.arange(_BROWS, dtype=jnp.int32).reshape(1, _BROWS)
    zrows = jnp.zeros((_RPT, 16), jnp.float32)
    zbins = jnp.zeros((_BROWS, 16), jnp.float32)
    znb = jnp.zeros((_NROWS, 16), jnp.float32)

    hf_pad, spins_pad = _ising_call(
        src, dst, h_pad, batch_pad, colors_pad, spins0_pad, u_flat,
        rowids, binids, zrows, zbins, znb)
    mask = spins_pad.reshape(-1)[:_N] > 0.0
    return (mask, hf_pad[:_N])


# speedup vs baseline: 60.4546x; 4.4398x over previous
import jax
import jax.numpy as jnp
from jax import lax
from jax.experimental import pallas as pl
from jax.experimental.pallas import tpu as pltpu
from jax.experimental.pallas import tpu_sc as plsc

_N = 10000
_E = 640000
_NG = 128
_NCOLORS = 4
_NITER = 5
_NSTEPS = _NITER * _NCOLORS

_NP = 10240
_TILES = 16
_EPT = _E // _TILES
_NPT = _NP // _TILES
_NROWS = _NP // 16
_RPT = _NROWS // _TILES
_BROWS = 32
_BPAD = 256


def _ising_body(src_hbm, dst_hbm, h_hbm, batch_hbm, colors_hbm, spins0_hbm,
                u_hbm, rowids_hbm, binids_hbm, zrows_hbm, zbins_hbm, znb_hbm,
                hf_out, spins_out,
                src_v, dst_v, spins_v, nb_v, h_v, batch_v, colors_v, hf_v,
                u_v, bins_v, binsred_v, mean_v, nbsl_v, rowids_v, binids_v,
                zrows_v, shared_nb, shared_sp, shared_bins):
    cid = lax.axis_index("c")
    sid = lax.axis_index("s")

    @pl.when(cid == 0)
    def _work():
        ebase = sid * _EPT
        nbase = sid * _NPT
        rbase = sid * _RPT
        lanes = lax.iota(jnp.int32, 16)

        pltpu.sync_copy(src_hbm.at[pl.ds(ebase, _EPT)], src_v)
        pltpu.sync_copy(dst_hbm.at[pl.ds(ebase, _EPT)], dst_v)
        pltpu.sync_copy(h_hbm.at[pl.ds(nbase, _NPT)], h_v)
        pltpu.sync_copy(batch_hbm.at[pl.ds(nbase, _NPT)], batch_v)
        pltpu.sync_copy(colors_hbm.at[pl.ds(nbase, _NPT)], colors_v)
        pltpu.sync_copy(spins0_hbm, spins_v)
        pltpu.sync_copy(rowids_hbm, rowids_v)
        pltpu.sync_copy(binids_hbm, binids_v)
        pltpu.sync_copy(zrows_hbm, zrows_v)
        pltpu.sync_copy(zbins_hbm, bins_v)
        for s in range(_NSTEPS):
            pltpu.sync_copy(u_hbm.at[pl.ds(s * _NP + nbase, _NPT)],
                            u_v.at[pl.ds(s * _NPT, _NPT)])

        ones = jnp.ones((16,), jnp.float32)

        def _rb(i, carry):
            b = batch_v[pl.ds(i * 16, 16)]
            hv = h_v[pl.ds(i * 16, 16)]
            hi = jnp.right_shift(b, 4)
            lo = jnp.bitwise_and(b, 15)
            plsc.addupdate_scatter(bins_v, [hi, lo], hv)
            plsc.addupdate_scatter(bins_v, [hi + 8, lo], ones)
            return carry
        lax.fori_loop(0, _NPT // 16, _rb, 0)

        @pl.when(sid == 0)
        def _zb():
            pltpu.sync_copy(zbins_hbm, shared_bins)
        plsc.subcore_barrier()
        pltpu.sync_copy(bins_v, shared_bins.at[binids_v.at[0]], add=True)
        plsc.subcore_barrier()
        pltpu.sync_copy(shared_bins, binsred_v)

        for i in range(_NG // 16):
            ri = jnp.full((16,), i, jnp.int32)
            sb = plsc.load_gather(binsred_v, [ri, lanes])
            cb = plsc.load_gather(binsred_v, [ri + 8, lanes])
            m = sb / jnp.maximum(cb, 1.0)
            plsc.store_scatter(mean_v, [ri, lanes], m)

        def _hb(i, carry):
            b = jnp.minimum(batch_v[pl.ds(i * 16, 16)], 127)
            m = plsc.load_gather(mean_v,
                                 [jnp.right_shift(b, 4),
                                  jnp.bitwise_and(b, 15)])
            hf_v[pl.ds(i * 16, 16)] = h_v[pl.ds(i * 16, 16)] - m
            return carry
        lax.fori_loop(0, _NPT // 16, _hb, 0)
        pltpu.sync_copy(hf_v, hf_out.at[pl.ds(nbase, _NPT)])

        for s in range(_NSTEPS):
            c = s % _NCOLORS
            pltpu.sync_copy(znb_hbm, nb_v)
            pltpu.sync_copy(zrows_v, shared_nb.at[pl.ds(rbase, _RPT)])
            plsc.subcore_barrier()

            def _eb(i, carry):
                sv = src_v[pl.ds(i * 16, 16)]
                dv = dst_v[pl.ds(i * 16, 16)]
                sp = plsc.load_gather(spins_v,
                                      [jnp.right_shift(sv, 4),
                                       jnp.bitwise_and(sv, 15)])
                plsc.addupdate_scatter(
                    nb_v, [jnp.right_shift(dv, 4), jnp.bitwise_and(dv, 15)],
                    sp)
                return carry
            lax.fori_loop(0, _EPT // 16, _eb, 0)

            for j in range(_NROWS // 128):
                pltpu.sync_copy(nb_v.at[pl.ds(j * 128, 128)],
                                shared_nb.at[rowids_v.at[j]], add=True)
            plsc.subcore_barrier()

            pltpu.sync_copy(shared_nb.at[pl.ds(rbase, _RPT)], nbsl_v)

            def _ub(i, carry):
                ri = jnp.full((16,), i, jnp.int32)
                nb16 = plsc.load_gather(nbsl_v, [ri, lanes])
                f = nb16 + hf_v[pl.ds(i * 16, 16)]
                p = 1.0 / (1.0 + jnp.exp(f * -20.0))
                u = u_v[pl.ds(s * _NPT + i * 16, 16)]
                newsp = jnp.where(u < p, 1.0, -1.0)
                col = colors_v[pl.ds(i * 16, 16)]
                grow = jnp.full((16,), rbase + i, jnp.int32)
                old = plsc.load_gather(spins_v, [grow, lanes])
                plsc.store_scatter(spins_v, [grow, lanes],
                                   jnp.where(col == c, newsp, old))
                return carry
            lax.fori_loop(0, _NPT // 16, _ub, 0)

            pltpu.sync_copy(spins_v.at[pl.ds(rbase, _RPT)],
                            shared_sp.at[pl.ds(rbase, _RPT)])
            plsc.subcore_barrier()
            pltpu.sync_copy(shared_sp, spins_v)

        pltpu.sync_copy(spins_v.at[pl.ds(rbase, _RPT)],
                        spins_out.at[pl.ds(rbase, _RPT)])


def _make_ising_call(interpret=False):
  return pl.kernel(
    _ising_body,
    out_type=(jax.ShapeDtypeStruct((_NP,), jnp.float32),
              jax.ShapeDtypeStruct((_NROWS, 16), jnp.float32)),
    mesh=plsc.VectorSubcoreMesh(core_axis_name="c", subcore_axis_name="s",
                                num_cores=2, num_subcores=16),
    interpret=interpret,
    compiler_params=pltpu.CompilerParams(needs_layout_passes=False,
                                         use_tc_tiling_on_sc=False),
    scratch_types=[
        pltpu.VMEM((_EPT,), jnp.int32),
        pltpu.VMEM((_EPT,), jnp.int32),
        pltpu.VMEM((_NROWS, 16), jnp.float32),
        pltpu.VMEM((_NROWS, 16), jnp.float32),
        pltpu.VMEM((_NPT,), jnp.float32),
        pltpu.VMEM((_NPT,), jnp.int32),
        pltpu.VMEM((_NPT,), jnp.int32),
        pltpu.VMEM((_NPT,), jnp.float32),
        pltpu.VMEM((_NSTEPS * _NPT,), jnp.float32),
        pltpu.VMEM((_BROWS, 16), jnp.float32),
        pltpu.VMEM((_BROWS, 16), jnp.float32),
        pltpu.VMEM((_NG // 16, 16), jnp.float32),
        pltpu.VMEM((_RPT, 16), jnp.float32),
        pltpu.VMEM((_NROWS // 128, 128), jnp.int32),
        pltpu.VMEM((1, _BROWS), jnp.int32),
        pltpu.VMEM((_RPT, 16), jnp.float32),
        pltpu.VMEM_SHARED((_NROWS, 16), jnp.float32),
        pltpu.VMEM_SHARED((_NROWS, 16), jnp.float32),
        pltpu.VMEM_SHARED((_BROWS, 16), jnp.float32),
    ],
  )


_ising_call = _make_ising_call()


_NWK = 32
_EPW = _E // _NWK
_ECH = 80
_NCHK = _EPW // _ECH
_RPS = _NP // 16


def _edge_body_fn(F):
    def body(x_hbm, ew_hbm, src_hbm, dst_hbm, z_hbm, out_hbm,
             src_v, dst_v, xg0, xg1, ew0, ew1, ms0, ms1,
             agg_sh, sg0, sg1):
        cid = lax.axis_index("c")
        sid = lax.axis_index("s")
        wid = sid * 2 + cid
        ebase = wid * _EPW
        xg = (xg0, xg1)
        ewb = (ew0, ew1)
        ms = (ms0, ms1)
        sg = (sg0, sg1)

        pltpu.sync_copy(src_hbm.at[wid], src_v)
        pltpu.sync_copy(dst_hbm.at[wid], dst_v)
        pltpu.sync_copy(z_hbm, agg_sh.at[pl.ds(sid * _RPS, _RPS)])
        plsc.subcore_barrier()

        for b in (0, 1):
            pltpu.async_copy(x_hbm.at[src_v.at[b]], xg[b], sg[b])
            pltpu.async_copy(ew_hbm.at[pl.ds(ebase + b * _ECH, _ECH)],
                             ewb[b], sg[b])

        def outer(i, carry):
            for b in (0, 1):
                k = i * 2 + b
                pltpu.make_async_copy(x_hbm.at[src_v.at[k]], xg[b],
                                      sg[b]).wait()
                pltpu.make_async_copy(
                    ew_hbm.at[pl.ds(ebase + k * _ECH, _ECH)], ewb[b],
                    sg[b]).wait()

                def row(r, c2):
                    for j in range(F // 16):
                        ms[b][r, pl.ds(j * 16, 16)] = jnp.maximum(
                            xg[b][r, pl.ds(j * 16, 16)]
                            + ewb[b][r, pl.ds(j * 16, 16)], 0.0)
                    return c2
                lax.fori_loop(0, _ECH, row, 0)

                pltpu.sync_copy(ms[b], agg_sh.at[dst_v.at[k]], add=True)

                @pl.when(k + 2 < _NCHK)
                def _prefetch():
                    pltpu.async_copy(x_hbm.at[src_v.at[k + 2]], xg[b], sg[b])
                    pltpu.async_copy(
                        ew_hbm.at[pl.ds(ebase + (k + 2) * _ECH, _ECH)],
                        ewb[b], sg[b])
            return carry
        lax.fori_loop(0, _NCHK // 2, outer, 0)

        plsc.subcore_barrier()
        pltpu.sync_copy(agg_sh.at[pl.ds(sid * _RPS, _RPS)],
                        out_hbm.at[pl.ds(cid * _NP + sid * _RPS, _RPS)])
    return body


def _make_edge_call(F, interpret=False):
    return pl.kernel(
        _edge_body_fn(F),
        out_type=jax.ShapeDtypeStruct((2 * _NP, F), jnp.float32),
        mesh=plsc.VectorSubcoreMesh(core_axis_name="c", subcore_axis_name="s",
                                    num_cores=2, num_subcores=16),
        interpret=interpret,
        compiler_params=pltpu.CompilerParams(needs_layout_passes=False,
                                             use_tc_tiling_on_sc=False),
        scratch_types=[
            pltpu.VMEM((_NCHK, _ECH), jnp.int32),
            pltpu.VMEM((_NCHK, _ECH), jnp.int32),
            pltpu.VMEM((_ECH, F), jnp.float32),
            pltpu.VMEM((_ECH, F), jnp.float32),
            pltpu.VMEM((_ECH, F), jnp.float32),
            pltpu.VMEM((_ECH, F), jnp.float32),
            pltpu.VMEM((_ECH, F), jnp.float32),
            pltpu.VMEM((_ECH, F), jnp.float32),
            pltpu.VMEM_SHARED((_NP, F), jnp.float32),
            pltpu.SemaphoreType.DMA,
            pltpu.SemaphoreType.DMA,
        ],
    )


_edge_call_64 = _make_edge_call(64)


def _edge_agg(x_pad, ew, srcw, dstw, zrows):
    F = x_pad.shape[1]
    if F > 64:
        return jnp.concatenate(
            [_edge_agg(x_pad[:, j:j + 64], ew[:, j:j + 64], srcw, dstw, zrows)
             for j in range(0, F, 64)], axis=1)
    out2 = _edge_call_64(x_pad, ew, srcw, dstw, zrows)
    return out2[:_NP] + out2[_NP:]


def _gine_sc(x_pad, ew, srcw, dstw, zrows, Wa, ba, Wb, bb):
    h = x_pad + _edge_agg(x_pad, ew, srcw, dstw, zrows)
    return jax.nn.relu(h @ Wa + ba) @ Wb + bb


def kernel(x, edge_index, edge_attr, batch, classification_model, colors, train,
           We1, W1a, b1a, W1b, b1b,
           We2, W2a, b2a, W2b, b2b,
           We3, W3a, b3a, W3b, b3b):
    src = edge_index[0]
    dst = edge_index[1]
    srcw = src.reshape(_NWK, _NCHK, _ECH)
    dstw = dst.reshape(_NWK, _NCHK, _ECH)
    zrows64 = jnp.zeros((_RPS, 64), jnp.float32)
    pad = _NP - _N
    x_pad = jnp.pad(x, ((0, pad), (0, 0)))
    h = _gine_sc(x_pad, edge_attr @ We1, srcw, dstw,
                 zrows64, W1a, b1a, W1b, b1b)
    h = _gine_sc(h, edge_attr @ We2, srcw, dstw,
                 zrows64, W2a, b2a, W2b, b2b)
    h = _gine_sc(h, edge_attr @ We3, srcw, dstw,
                 zrows64, W3a, b3a, W3b, b3b)
    hflat = h.reshape(-1)

    key = jax.random.key(7)
    u0 = jax.random.uniform(key, (_N,))
    spins0 = jnp.where(u0 < 0.5, 1.0, -1.0).astype(jnp.float32)
    urows = [jax.random.uniform(jax.random.fold_in(key, s + 1), (_N,))
             for s in range(_NSTEPS)]
    u_flat = jnp.concatenate(
        [jnp.pad(r, (0, pad)) for r in urows]).astype(jnp.float32)

    h_pad = hflat
    batch_pad = jnp.pad(batch, (0, pad), constant_values=_BPAD)
    colors_pad = jnp.pad(colors, (0, pad), constant_values=-1)
    spins0_pad = jnp.pad(spins0, (0, pad)).reshape(_NROWS, 16)
    rowids = jnp.arange(_NROWS, dtype=jnp.int32).reshape(_NROWS // 128, 128)
    binids = jnp.arange(_BROWS, dtype=jnp.int32).reshape(1, _BROWS)
    zrows = jnp.zeros((_RPT, 16), jnp.float32)
    zbins = jnp.zeros((_BROWS, 16), jnp.float32)
    znb = jnp.zeros((_NROWS, 16), jnp.float32)

    hf_pad, spins_pad = _ising_call(
        src, dst, h_pad, batch_pad, colors_pad, spins0_pad, u_flat,
        rowids, binids, zrows, zbins, znb)
    mask = spins_pad.reshape(-1)[:_N] > 0.0
    return (mask, hf_pad[:_N])
